# baseline (speedup 1.0000x reference)
import functools

import jax
import jax.numpy as jnp
from jax import lax
from jax.experimental import pallas as pl
from jax.experimental.pallas import tpu as pltpu
from jax.experimental.pallas import tpu_sc as plsc

_N = 100000
_E = 6400000
_IN = 4
_HID = 64
_OUT = 128
_G = 512

_NC = 2
_NS = 16
_NW = _NC * _NS
_CHUNK = 125
_GRP = 8
_RPT = 6256
_NPAD = _RPT * _NS
_CPW = 1600
_NGRP = _CPW // _GRP
_DP = 8


def _sc_agg_body(x_hbm, src_hbm, dst_hbm, zrows_hbm, out_hbm,
                 isrc0, idst0, rows0, isrc1, idst1, rows1, acc,
                 sem0, sem1):
    c = lax.axis_index("c")
    s = lax.axis_index("s")
    w = c * _NS + s
    bufs = ((isrc0, idst0, rows0, sem0), (isrc1, idst1, rows1, sem1))

    def stage_and_fire(g, buf):
        bsrc, bdst, brows, bsem = buf
        pltpu.sync_copy(src_hbm.at[w, pl.ds(g * _GRP, _GRP)], bsrc)
        pltpu.sync_copy(dst_hbm.at[w, pl.ds(g * _GRP, _GRP)], bdst)
        for j in range(_GRP):
            pltpu.async_copy(x_hbm.at[bsrc.at[j]],
                             brows.at[pl.ds(j * _CHUNK, _CHUNK)], bsem)

    def drain(buf):
        _, _, brows, bsem = buf
        pltpu.make_async_copy(x_hbm.at[pl.ds(0, _GRP * _CHUNK)],
                              brows, bsem).wait()

    def scatter(buf):
        _, bdst, brows, _ = buf
        for j in range(_GRP):
            pltpu.sync_copy(brows.at[pl.ds(j * _CHUNK, _CHUNK)],
                            acc.at[bdst.at[j]], add=True)

    pltpu.sync_copy(zrows_hbm, acc.at[pl.ds(s * _RPT, _RPT)])
    plsc.subcore_barrier()

    stage_and_fire(0, bufs[0])

    def pair(ii, carry):
        for pp in range(2):
            g = 2 * ii + pp
            nxt = jnp.where(g + 1 < _NGRP, g + 1, 0)
            stage_and_fire(nxt, bufs[1 - pp])
            drain(bufs[pp])
            scatter(bufs[pp])
        return carry

    lax.fori_loop(0, _NGRP // 2, pair, 0)
    drain(bufs[0])

    plsc.subcore_barrier()
    pltpu.sync_copy(acc.at[pl.ds(s * _RPT, _RPT)],
                    out_hbm.at[c, pl.ds(s * _RPT, _RPT)])


@functools.cache
def _sc_agg():
    return pl.kernel(
        _sc_agg_body,
        out_type=jax.ShapeDtypeStruct((_NC, _NPAD, _DP), jnp.float32),
        mesh=plsc.VectorSubcoreMesh(core_axis_name="c", subcore_axis_name="s",
                                    num_cores=_NC, num_subcores=_NS),
        scratch_types=[
            pltpu.VMEM((_GRP, _CHUNK), jnp.int32),
            pltpu.VMEM((_GRP, _CHUNK), jnp.int32),
            pltpu.VMEM((_GRP * _CHUNK, _DP), jnp.float32),
            pltpu.VMEM((_GRP, _CHUNK), jnp.int32),
            pltpu.VMEM((_GRP, _CHUNK), jnp.int32),
            pltpu.VMEM((_GRP * _CHUNK, _DP), jnp.float32),
            pltpu.VMEM_SHARED((_NPAD, _DP), jnp.float32),
            pltpu.SemaphoreType.DMA,
            pltpu.SemaphoreType.DMA,
        ],
        compiler_params=pltpu.CompilerParams(use_tc_tiling_on_sc=False),
    )


_BN = 2000
_NBLK = _N // _BN


def _tc_body(x_ref, a0_ref, a1_ref, b_ref, w1_ref, b1_ref,
             w2_ref, b2_ref, w3_ref, b3_ref, out_ref, acc_ref):
    i = pl.program_id(0)
    s = x_ref[...] + a0_ref[...] + a1_ref[...]
    h = b1_ref[...] + s[:, 0:1] * w1_ref[0:1, :]
    for k in range(1, _IN):
        h = h + s[:, k : k + 1] * w1_ref[k : k + 1, :]
    h = jnp.maximum(h, 0.0)
    hh = jnp.concatenate(
        [h, jnp.ones((_BN, 1), jnp.float32),
         jnp.zeros((_BN, _OUT - _HID - 1), jnp.float32)], axis=1)
    bt = b_ref[0]
    gid = lax.broadcasted_iota(jnp.int32, (_G, _BN), 0)
    onehot_t = (gid == bt).astype(jnp.float32)
    contrib = jnp.dot(onehot_t, hh, preferred_element_type=jnp.float32)

    @pl.when(i == 0)
    def _():
        acc_ref[...] = jnp.zeros_like(acc_ref)

    acc_ref[...] += contrib

    @pl.when(i == _NBLK - 1)
    def _():
        accv = acc_ref[...]
        counts = jnp.maximum(accv[:, _HID : _HID + 1], 1.0)
        pooled = accv[:, :_HID] / counts
        w23 = jnp.dot(w2_ref[...], w3_ref[...],
                      preferred_element_type=jnp.float32)
        b23 = jnp.dot(b2_ref[...], w3_ref[...],
                      preferred_element_type=jnp.float32) + b3_ref[...]
        out_ref[...] = jnp.dot(pooled, w23,
                               preferred_element_type=jnp.float32) + b23


def _tc_pool(x, agg0, agg1, batch3, W1, b1, W2, b2, W3, b3):
    row_spec = pl.BlockSpec((_BN, _IN), lambda i: (i, 0))
    full = lambda shape: pl.BlockSpec(shape, lambda i: (0,) * len(shape))
    return pl.pallas_call(
        _tc_body,
        grid=(_NBLK,),
        in_specs=[
            row_spec, row_spec, row_spec,
            pl.BlockSpec((1, 1, _BN), lambda i: (i, 0, 0)),
            full((_IN, _HID)), full((1, _HID)),
            full((_HID, _HID)), full((1, _HID)),
            full((_HID, _OUT)), full((1, _OUT)),
        ],
        out_specs=pl.BlockSpec((_G, _OUT), lambda i: (0, 0)),
        out_shape=jax.ShapeDtypeStruct((_G, _OUT), jnp.float32),
        scratch_shapes=[pltpu.VMEM((_G, _OUT), jnp.float32)],
    )(x, agg0, agg1, batch3, W1, b1, W2, b2, W3, b3)


def kernel(x, edge_index, batch, W1, b1, W2, b2, W3, b3):
    src = edge_index[0]
    dst = edge_index[1]
    src_p = src.reshape(_NW, _CPW, _CHUNK)
    dst_p = dst.reshape(_NW, _CPW, _CHUNK)
    x_pad = jnp.zeros((_NPAD, _DP), jnp.float32).at[:_N, :_IN].set(x)
    zrows = jnp.zeros((_RPT, _DP), jnp.float32)
    agg2 = _sc_agg()(x_pad, src_p, dst_p, zrows)
    batch3 = batch.reshape(_NBLK, 1, _BN)
    return _tc_pool(x, agg2[0, :_N, :_IN], agg2[1, :_N, :_IN], batch3,
                    W1, b1[None], W2, b2[None], W3, b3[None])


# speedup vs baseline: 2.0142x; 2.0142x over previous
import functools

import jax
import jax.numpy as jnp
from jax import lax
from jax.experimental import pallas as pl
from jax.experimental.pallas import tpu as pltpu
from jax.experimental.pallas import tpu_sc as plsc

_N = 100000
_E = 6400000
_IN = 4
_HID = 64
_OUT = 128
_G = 512

_NC = 2
_NS = 16
_NW = _NC * _NS
_CHUNK = 128
_GRP = 8
_RPT = 6256
_NPAD = _RPT * _NS
_KCH = _E // _CHUNK
_NFULL = 195
_TBASE = _NFULL * _GRP
_DP = 8


def _sc_agg_body(x_hbm, edges_hbm, zrows_hbm, out_hbm,
                 isrc0, idst0, rows0, isrc1, idst1, rows1, acc,
                 sem0, sem1):
    c = lax.axis_index("c")
    s = lax.axis_index("s")
    w = c * _NS + s
    base = w * 1562 + jnp.minimum(w, 16)
    cnt = jnp.where(w < 16, 1563, 1562)
    bufs = ((isrc0, idst0, rows0, sem0), (isrc1, idst1, rows1, sem1))

    def stage_and_fire(g, buf):
        bsrc, bdst, brows, bsem = buf
        row = base + g * _GRP
        pltpu.sync_copy(edges_hbm.at[0, pl.ds(row, _GRP)], bsrc)
        pltpu.sync_copy(edges_hbm.at[1, pl.ds(row, _GRP)], bdst)
        for j in range(_GRP):
            pltpu.async_copy(x_hbm.at[bsrc.at[j]],
                             brows.at[pl.ds(j * _CHUNK, _CHUNK)], bsem)

    def drain(buf):
        _, _, brows, bsem = buf
        pltpu.make_async_copy(x_hbm.at[pl.ds(0, _GRP * _CHUNK)],
                              brows, bsem).wait()

    def scatter(buf):
        _, bdst, brows, _ = buf
        for j in range(_GRP):
            pltpu.sync_copy(brows.at[pl.ds(j * _CHUNK, _CHUNK)],
                            acc.at[bdst.at[j]], add=True)

    pltpu.sync_copy(zrows_hbm, acc.at[pl.ds(s * _RPT, _RPT)])
    plsc.subcore_barrier()

    stage_and_fire(0, bufs[0])

    def pair(ii, carry):
        for pp in range(2):
            g = 2 * ii + pp
            stage_and_fire(g + 1, bufs[1 - pp])
            drain(bufs[pp])
            scatter(bufs[pp])
        return carry

    lax.fori_loop(0, _NFULL // 2, pair, 0)
    drain(bufs[0])
    scatter(bufs[0])

    for j in range(3):
        @pl.when(j < cnt - _TBASE)
        def _():
            row = base + _TBASE + j
            pltpu.sync_copy(edges_hbm.at[0, row], isrc1.at[0])
            pltpu.sync_copy(edges_hbm.at[1, row], idst1.at[0])
            pltpu.async_copy(x_hbm.at[isrc1.at[0]],
                             rows1.at[pl.ds(0, _CHUNK)], sem1).wait()
            pltpu.sync_copy(rows1.at[pl.ds(0, _CHUNK)],
                            acc.at[idst1.at[0]], add=True)

    plsc.subcore_barrier()
    pltpu.sync_copy(acc.at[pl.ds(s * _RPT, _RPT)],
                    out_hbm.at[c, pl.ds(s * _RPT, _RPT)])


@functools.cache
def _sc_agg():
    return pl.kernel(
        _sc_agg_body,
        out_type=jax.ShapeDtypeStruct((_NC, _NPAD, _DP), jnp.float32),
        mesh=plsc.VectorSubcoreMesh(core_axis_name="c", subcore_axis_name="s",
                                    num_cores=_NC, num_subcores=_NS),
        scratch_types=[
            pltpu.VMEM((_GRP, _CHUNK), jnp.int32),
            pltpu.VMEM((_GRP, _CHUNK), jnp.int32),
            pltpu.VMEM((_GRP * _CHUNK, _DP), jnp.float32),
            pltpu.VMEM((_GRP, _CHUNK), jnp.int32),
            pltpu.VMEM((_GRP, _CHUNK), jnp.int32),
            pltpu.VMEM((_GRP * _CHUNK, _DP), jnp.float32),
            pltpu.VMEM_SHARED((_NPAD, _DP), jnp.float32),
            pltpu.SemaphoreType.DMA,
            pltpu.SemaphoreType.DMA,
        ],
        compiler_params=pltpu.CompilerParams(use_tc_tiling_on_sc=False),
    )


_BN = 2000
_NBLK = _N // _BN


def _tc_body(x_ref, a0_ref, a1_ref, b_ref, w1_ref, b1_ref,
             w2_ref, b2_ref, w3_ref, b3_ref, out_ref, acc_ref):
    i = pl.program_id(0)
    s = x_ref[...] + a0_ref[0, :, :_IN] + a1_ref[0, :, :_IN]
    h = b1_ref[...] + s[:, 0:1] * w1_ref[0:1, :]
    for k in range(1, _IN):
        h = h + s[:, k : k + 1] * w1_ref[k : k + 1, :]
    h = jnp.maximum(h, 0.0)
    hh = jnp.concatenate(
        [h, jnp.ones((_BN, 1), jnp.float32),
         jnp.zeros((_BN, _OUT - _HID - 1), jnp.float32)], axis=1)
    bt = b_ref[0]
    gid = lax.broadcasted_iota(jnp.int32, (_G, _BN), 0)
    onehot_t = (gid == bt).astype(jnp.float32)
    contrib = jnp.dot(onehot_t, hh, preferred_element_type=jnp.float32)

    @pl.when(i == 0)
    def _():
        acc_ref[...] = jnp.zeros_like(acc_ref)

    acc_ref[...] += contrib

    @pl.when(i == _NBLK - 1)
    def _():
        accv = acc_ref[...]
        counts = jnp.maximum(accv[:, _HID : _HID + 1], 1.0)
        pooled = accv[:, :_HID] / counts
        w23 = jnp.dot(w2_ref[...], w3_ref[...],
                      preferred_element_type=jnp.float32)
        b23 = jnp.dot(b2_ref[...], w3_ref[...],
                      preferred_element_type=jnp.float32) + b3_ref[...]
        out_ref[...] = jnp.dot(pooled, w23,
                               preferred_element_type=jnp.float32) + b23


def _tc_pool(x, agg2, batch3, W1, b1, W2, b2, W3, b3):
    row_spec = pl.BlockSpec((_BN, _IN), lambda i: (i, 0))
    agg0_spec = pl.BlockSpec((1, _BN, _DP), lambda i: (0, i, 0))
    agg1_spec = pl.BlockSpec((1, _BN, _DP), lambda i: (1, i, 0))
    full = lambda shape: pl.BlockSpec(shape, lambda i: (0,) * len(shape))
    return pl.pallas_call(
        _tc_body,
        grid=(_NBLK,),
        in_specs=[
            row_spec, agg0_spec, agg1_spec,
            pl.BlockSpec((1, 1, _BN), lambda i: (i, 0, 0)),
            full((_IN, _HID)), full((1, _HID)),
            full((_HID, _HID)), full((1, _HID)),
            full((_HID, _OUT)), full((1, _OUT)),
        ],
        out_specs=pl.BlockSpec((_G, _OUT), lambda i: (0, 0)),
        out_shape=jax.ShapeDtypeStruct((_G, _OUT), jnp.float32),
        scratch_shapes=[pltpu.VMEM((_G, _OUT), jnp.float32)],
    )(x, agg2, agg2, batch3, W1, b1, W2, b2, W3, b3)


def kernel(x, edge_index, batch, W1, b1, W2, b2, W3, b3):
    edge3 = edge_index.reshape(2, _KCH, _CHUNK)
    x_pad = jnp.zeros((_NPAD, _DP), jnp.float32).at[:_N, :_IN].set(x)
    zrows = jnp.zeros((_RPT, _DP), jnp.float32)
    agg2 = _sc_agg()(x_pad, edge3, zrows)
    batch3 = batch.reshape(_NBLK, 1, _BN)
    return _tc_pool(x, agg2, batch3,
                    W1, b1[None], W2, b2[None], W3, b3[None])
